# baseline (speedup 1.0000x reference)
import jax
import jax.numpy as jnp
from jax.experimental import pallas as pl
from jax.experimental.pallas import tpu as pltpu

B, H, S, D = 1, 16, 2048, 128
BQ = 512


def _attn_block(q_ref, k_ref, v_ref, o_ref):
    q = q_ref[0]
    k = k_ref[0]
    s = jax.lax.dot_general(q, k, (((1,), (1,)), ((), ())),
                            preferred_element_type=jnp.float32)
    m = jnp.max(s, axis=1, keepdims=True)
    p = jnp.exp(s - m)
    l = jnp.sum(p, axis=1, keepdims=True)
    o = jax.lax.dot_general(p.astype(jnp.bfloat16), v_ref[0],
                            (((1,), (0,)), ((), ())),
                            preferred_element_type=jnp.float32)
    o_ref[0] = o / l


def kernel(query, key, value):
    scale = D ** (-0.5)
    q = (query.reshape(H, S, D) * scale).astype(jnp.bfloat16)
    k = key.reshape(H, S, D).astype(jnp.bfloat16)
    v = value.reshape(H, S, D).astype(jnp.bfloat16)
    out = pl.pallas_call(
        _attn_block,
        grid=(H, S // BQ),
        in_specs=[
            pl.BlockSpec((1, BQ, D), lambda h, i: (h, i, 0)),
            pl.BlockSpec((1, S, D), lambda h, i: (h, 0, 0)),
            pl.BlockSpec((1, S, D), lambda h, i: (h, 0, 0)),
        ],
        out_specs=pl.BlockSpec((1, BQ, D), lambda h, i: (h, i, 0)),
        out_shape=jax.ShapeDtypeStruct((H, S, D), jnp.float32),
        compiler_params=pltpu.CompilerParams(
            dimension_semantics=("parallel", "parallel"),
        ),
    )(q, k, v)
    return out.reshape(B, H, S, D)


# speedup vs baseline: 1.1292x; 1.1292x over previous
import functools

import jax
import jax.numpy as jnp
from jax.experimental import pallas as pl
from jax.experimental.pallas import tpu as pltpu

B, H, S, D = 1, 16, 2048, 128
BQ = 512


def _attn_block(q_ref, k_ref, v_ref, o_ref, kb_ref, vb_ref, *, scale):
    @pl.when(pl.program_id(1) == 0)
    def _cast_kv():
        kb_ref[...] = k_ref[0].astype(jnp.bfloat16)
        vb_ref[...] = v_ref[0].astype(jnp.bfloat16)

    q = (q_ref[0] * scale).astype(jnp.bfloat16)
    s = jax.lax.dot_general(q, kb_ref[...], (((1,), (1,)), ((), ())),
                            preferred_element_type=jnp.float32)
    m = jnp.max(s, axis=1, keepdims=True)
    p = jnp.exp(s - m)
    l = jnp.sum(p, axis=1, keepdims=True)
    o = jax.lax.dot_general(p.astype(jnp.bfloat16), vb_ref[...],
                            (((1,), (0,)), ((), ())),
                            preferred_element_type=jnp.float32)
    o_ref[0] = o / l


def kernel(query, key, value):
    scale = D ** (-0.5)
    q = query.reshape(H, S, D)
    k = key.reshape(H, S, D)
    v = value.reshape(H, S, D)
    out = pl.pallas_call(
        functools.partial(_attn_block, scale=scale),
        grid=(H, S // BQ),
        in_specs=[
            pl.BlockSpec((1, BQ, D), lambda h, i: (h, i, 0)),
            pl.BlockSpec((1, S, D), lambda h, i: (h, 0, 0)),
            pl.BlockSpec((1, S, D), lambda h, i: (h, 0, 0)),
        ],
        out_specs=pl.BlockSpec((1, BQ, D), lambda h, i: (h, i, 0)),
        out_shape=jax.ShapeDtypeStruct((H, S, D), jnp.float32),
        scratch_shapes=[
            pltpu.VMEM((S, D), jnp.bfloat16),
            pltpu.VMEM((S, D), jnp.bfloat16),
        ],
        compiler_params=pltpu.CompilerParams(
            dimension_semantics=("parallel", "arbitrary"),
        ),
    )(q, k, v)
    return out.reshape(B, H, S, D)


# speedup vs baseline: 1.5835x; 1.4023x over previous
import functools

import jax
import jax.numpy as jnp
from jax.experimental import pallas as pl
from jax.experimental.pallas import tpu as pltpu

B, H, S, D = 1, 16, 2048, 128
BQ = 1024
SUB = 512


def _sub_attn(q, kb, vb, scale):
    qb = (q * scale).astype(jnp.bfloat16)
    s = jax.lax.dot_general(qb, kb, (((1,), (1,)), ((), ())),
                            preferred_element_type=jnp.float32)
    m = jnp.max(s, axis=1, keepdims=True)
    p = jnp.exp(s - m)
    l = jnp.sum(p, axis=1, keepdims=True)
    o = jax.lax.dot_general(p.astype(jnp.bfloat16), vb,
                            (((1,), (0,)), ((), ())),
                            preferred_element_type=jnp.float32)
    return o / l


def _attn_block(q_ref, k_ref, v_ref, o_ref, kb_ref, vb_ref, *, scale):
    @pl.when(pl.program_id(1) == 0)
    def _cast_kv():
        kb_ref[...] = k_ref[0].astype(jnp.bfloat16)
        vb_ref[...] = v_ref[0].astype(jnp.bfloat16)

    kb = kb_ref[...]
    vb = vb_ref[...]
    o_ref[0, :SUB] = _sub_attn(q_ref[0, :SUB], kb, vb, scale)
    o_ref[0, SUB:] = _sub_attn(q_ref[0, SUB:], kb, vb, scale)


def kernel(query, key, value):
    scale = D ** (-0.5)
    q = query.reshape(H, S, D)
    k = key.reshape(H, S, D)
    v = value.reshape(H, S, D)
    out = pl.pallas_call(
        functools.partial(_attn_block, scale=scale),
        grid=(H, S // BQ),
        in_specs=[
            pl.BlockSpec((1, BQ, D), lambda h, i: (h, i, 0)),
            pl.BlockSpec((1, S, D), lambda h, i: (h, 0, 0)),
            pl.BlockSpec((1, S, D), lambda h, i: (h, 0, 0)),
        ],
        out_specs=pl.BlockSpec((1, BQ, D), lambda h, i: (h, i, 0)),
        out_shape=jax.ShapeDtypeStruct((H, S, D), jnp.float32),
        scratch_shapes=[
            pltpu.VMEM((S, D), jnp.bfloat16),
            pltpu.VMEM((S, D), jnp.bfloat16),
        ],
        compiler_params=pltpu.CompilerParams(
            dimension_semantics=("parallel", "arbitrary"),
        ),
    )(q, k, v)
    return out.reshape(B, H, S, D)


# speedup vs baseline: 2.0072x; 1.2676x over previous
import functools

import jax
import jax.numpy as jnp
from jax.experimental import pallas as pl
from jax.experimental.pallas import tpu as pltpu

B, H, S, D = 1, 16, 2048, 128
BQ = 2048
SUB = 512


def _sub_attn(q, kb, vb, scale):
    qb = (q * scale).astype(jnp.bfloat16)
    s = jax.lax.dot_general(qb, kb, (((1,), (1,)), ((), ())),
                            preferred_element_type=jnp.float32)
    m = jnp.max(s, axis=1, keepdims=True)
    p = jnp.exp(s - m)
    l = jnp.sum(p, axis=1, keepdims=True)
    o = jax.lax.dot_general(p.astype(jnp.bfloat16), vb,
                            (((1,), (0,)), ((), ())),
                            preferred_element_type=jnp.float32)
    return o / l


def _attn_block(q_ref, k_ref, v_ref, o_ref, kb_ref, vb_ref, *, scale):
    @pl.when(pl.program_id(1) == 0)
    def _cast_kv():
        kb_ref[...] = k_ref[0].astype(jnp.bfloat16)
        vb_ref[...] = v_ref[0].astype(jnp.bfloat16)

    kb = kb_ref[...]
    vb = vb_ref[...]
    for j in range(BQ // SUB):
        o_ref[0, j * SUB:(j + 1) * SUB] = _sub_attn(
            q_ref[0, j * SUB:(j + 1) * SUB], kb, vb, scale)


def kernel(query, key, value):
    scale = D ** (-0.5)
    q = query.reshape(H, S, D)
    k = key.reshape(H, S, D)
    v = value.reshape(H, S, D)
    out = pl.pallas_call(
        functools.partial(_attn_block, scale=scale),
        grid=(H, S // BQ),
        in_specs=[
            pl.BlockSpec((1, BQ, D), lambda h, i: (h, i, 0)),
            pl.BlockSpec((1, S, D), lambda h, i: (h, 0, 0)),
            pl.BlockSpec((1, S, D), lambda h, i: (h, 0, 0)),
        ],
        out_specs=pl.BlockSpec((1, BQ, D), lambda h, i: (h, i, 0)),
        out_shape=jax.ShapeDtypeStruct((H, S, D), jnp.float32),
        scratch_shapes=[
            pltpu.VMEM((S, D), jnp.bfloat16),
            pltpu.VMEM((S, D), jnp.bfloat16),
        ],
        compiler_params=pltpu.CompilerParams(
            dimension_semantics=("parallel", "arbitrary"),
        ),
    )(q, k, v)
    return out.reshape(B, H, S, D)
